# Initial kernel scaffold; baseline (speedup 1.0000x reference)
import functools

import jax
import jax.numpy as jnp
from jax import lax
from jax.experimental import pallas as pl
from jax.experimental.pallas import tpu as pltpu
from jax.experimental.pallas import tpu_sc as plsc

N_NODES = 10000
DIM = 128
NUM_EDGES = 320000
NUM_ITERS = 5
TELEPORT = 0.8

NC = 2
NS = 16
NW = NC * NS
CH = 128
NCHUNKS = 2528
EPAD = NCHUNKS * CH
CPT = NCHUNKS // NW
NP = 10240
RPT = NP // NS
HL = 16

_SC_MESH = plsc.VectorSubcoreMesh(core_axis_name="c", subcore_axis_name="s")


@functools.partial(
    pl.kernel,
    out_type=jax.ShapeDtypeStruct((NC, NP, HL), jnp.float32),
    mesh=_SC_MESH,
    scratch_types=[
        pltpu.VMEM((CH, HL), jnp.float32),
        pltpu.VMEM((CH,), jnp.int32),
        pltpu.VMEM_SHARED((NP, HL), jnp.float32),
    ],
)
def _sc_degree(col2d_hbm, hist_hbm, src_v, idx_v, hist_sh):
    c = lax.axis_index("c")
    s = lax.axis_index("s")
    tid = c * NS + s

    zero16 = jnp.zeros((HL,), jnp.float32)

    def zfill(r, _):
        src_v[r, :] = zero16
        return 0

    lax.fori_loop(0, CH, zfill, 0)
    for b in range(RPT // CH):
        pltpu.sync_copy(src_v, hist_sh.at[pl.ds(s * RPT + b * CH, CH)])
    plsc.subcore_barrier()

    one_hot = (lax.iota(jnp.int32, (HL,)) == 0).astype(jnp.float32)

    def ofill(r, _):
        src_v[r, :] = one_hot
        return 0

    lax.fori_loop(0, CH, ofill, 0)

    def chunk(i, _):
        ci = tid * CPT + i
        pltpu.sync_copy(col2d_hbm.at[ci], idx_v)
        pltpu.sync_copy(src_v, hist_sh.at[idx_v], add=True)
        return 0

    lax.fori_loop(0, CPT, chunk, 0)
    plsc.subcore_barrier()
    pltpu.sync_copy(hist_sh.at[pl.ds(s * RPT, RPT)],
                    hist_hbm.at[c, pl.ds(s * RPT, RPT)])


@functools.partial(
    pl.kernel,
    out_type=jax.ShapeDtypeStruct((NC, NP, DIM), jnp.float32),
    mesh=_SC_MESH,
    scratch_types=[
        pltpu.VMEM((CH, DIM), jnp.float32),
        pltpu.VMEM((CH,), jnp.int32),
        pltpu.VMEM((CH,), jnp.int32),
        pltpu.VMEM_SHARED((NP, DIM), jnp.float32),
        pltpu.SemaphoreType.DMA,
    ],
)
def _sc_propagate(row2d_hbm, col2d_hbm, g_hbm, out_hbm,
                  rows_v, ridx_v, cidx_v, acc_sh, sem):
    c = lax.axis_index("c")
    s = lax.axis_index("s")
    tid = c * NS + s

    zero16 = jnp.zeros((16,), jnp.float32)

    def zfill(r, _):
        for j in range(DIM // 16):
            rows_v[r, pl.ds(j * 16, 16)] = zero16
        return 0

    lax.fori_loop(0, CH, zfill, 0)
    for b in range(RPT // CH):
        pltpu.sync_copy(rows_v, acc_sh.at[pl.ds(s * RPT + b * CH, CH)])
    plsc.subcore_barrier()

    def chunk(i, _):
        ci = tid * CPT + i
        pltpu.sync_copy(row2d_hbm.at[ci], ridx_v)
        pltpu.sync_copy(col2d_hbm.at[ci], cidx_v)
        pltpu.async_copy(g_hbm.at[ridx_v], rows_v, sem).wait()
        pltpu.sync_copy(rows_v, acc_sh.at[cidx_v], add=True)
        return 0

    lax.fori_loop(0, CPT, chunk, 0)
    plsc.subcore_barrier()
    pltpu.sync_copy(acc_sh.at[pl.ds(s * RPT, RPT)],
                    out_hbm.at[c, pl.ds(s * RPT, RPT)])


RB = 500


def _deg_dis(hist):
    deg = (jnp.sum(hist[0], axis=-1, keepdims=True)
           + jnp.sum(hist[1], axis=-1, keepdims=True) + 1.0)
    return lax.rsqrt(deg)


def _tc_init_body(hist_ref, x_ref, g_ref):
    dis = _deg_dis(hist_ref[...])
    g_ref[...] = dis * x_ref[...]


def _tc_update_body(hist_ref, s_ref, g_ref, x_ref, o_ref):
    dis = _deg_dis(hist_ref[...])
    su = s_ref[0] + s_ref[1] + g_ref[...]
    o_ref[...] = ((1.0 - TELEPORT) * dis * dis) * su + (TELEPORT * dis) * x_ref[...]


def _tc_final_body(hist_ref, s_ref, g_ref, x_ref, o_ref):
    dis = _deg_dis(hist_ref[...])
    su = s_ref[0] + s_ref[1] + g_ref[...]
    o_ref[...] = ((1.0 - TELEPORT) * dis) * su + TELEPORT * x_ref[...]


_HIST_SPEC = pl.BlockSpec((NC, RB, HL), lambda i: (0, i, 0))
_S_SPEC = pl.BlockSpec((NC, RB, DIM), lambda i: (0, i, 0))
_ROW_SPEC = pl.BlockSpec((RB, DIM), lambda i: (i, 0))
_OUT_SDS = jax.ShapeDtypeStruct((N_NODES, DIM), jnp.float32)
_GRID = (N_NODES // RB,)

_tc_init = pl.pallas_call(
    _tc_init_body,
    grid=_GRID,
    in_specs=[_HIST_SPEC, _ROW_SPEC],
    out_specs=_ROW_SPEC,
    out_shape=_OUT_SDS,
)

_tc_update = pl.pallas_call(
    _tc_update_body,
    grid=_GRID,
    in_specs=[_HIST_SPEC, _S_SPEC, _ROW_SPEC, _ROW_SPEC],
    out_specs=_ROW_SPEC,
    out_shape=_OUT_SDS,
)

_tc_final = pl.pallas_call(
    _tc_final_body,
    grid=_GRID,
    in_specs=[_HIST_SPEC, _S_SPEC, _ROW_SPEC, _ROW_SPEC],
    out_specs=_ROW_SPEC,
    out_shape=_OUT_SDS,
)


def kernel(x, edge_index):
    row = edge_index[0]
    col = edge_index[1]
    pad = EPAD - NUM_EDGES
    rowp = jnp.concatenate([row, jnp.zeros((pad,), row.dtype)]).reshape(NCHUNKS, CH)
    colp = jnp.concatenate([col, jnp.full((pad,), N_NODES, col.dtype)]).reshape(NCHUNKS, CH)

    hist = _sc_degree(colp)
    g = _tc_init(hist, x)
    for it in range(NUM_ITERS):
        s_parts = _sc_propagate(rowp, colp, g)
        if it < NUM_ITERS - 1:
            g = _tc_update(hist, s_parts, g, x)
        else:
            g = _tc_final(hist, s_parts, g, x)
    return g


# speedup vs baseline: 7.9231x; 7.9231x over previous
import functools

import jax
import jax.numpy as jnp
from jax import lax
from jax.experimental import pallas as pl
from jax.experimental.pallas import tpu as pltpu
from jax.experimental.pallas import tpu_sc as plsc

N_NODES = 10000
DIM = 128
NUM_EDGES = 320000
NUM_ITERS = 5
TELEPORT = 0.8

NC = 2
NS = 16
NW = NC * NS
CH = 128
NCHUNKS = 2528
EPAD = NCHUNKS * CH
CPT = NCHUNKS // NW
NP = 10240
RPT = NP // NS
HL = 128

_SC_MESH = plsc.VectorSubcoreMesh(
    core_axis_name="c", subcore_axis_name="s", num_cores=NC, num_subcores=NS)


def _sc_degree_body(col2d_hbm, hist_hbm, src_v, idx_v, hist_sh):
    c = lax.axis_index("c")
    s = lax.axis_index("s")
    tid = c * NS + s

    zero16 = jnp.zeros((16,), jnp.float32)

    def zfill(r, _):
        for j in range(HL // 16):
            src_v[r, pl.ds(j * 16, 16)] = zero16
        return 0

    lax.fori_loop(0, CH, zfill, 0)
    for b in range(RPT // CH):
        pltpu.sync_copy(src_v, hist_sh.at[pl.ds(s * RPT + b * CH, CH)])
    plsc.subcore_barrier()

    ones16 = jnp.ones((16,), jnp.float32)

    def ofill(r, _):
        for j in range(HL // 16):
            src_v[r, pl.ds(j * 16, 16)] = ones16
        return 0

    lax.fori_loop(0, CH, ofill, 0)

    def chunk(i, _):
        ci = tid * CPT + i
        pltpu.sync_copy(col2d_hbm.at[ci], idx_v)
        pltpu.sync_copy(src_v, hist_sh.at[idx_v], add=True)
        return 0

    lax.fori_loop(0, CPT, chunk, 0)
    plsc.subcore_barrier()
    pltpu.sync_copy(hist_sh.at[pl.ds(s * RPT, RPT)],
                    hist_hbm.at[c, pl.ds(s * RPT, RPT)])


def _sc_propagate_body(row2d_hbm, col2d_hbm, g_hbm, out_hbm,
                  rows_v, ridx_v, cidx_v, acc_sh, sem):
    c = lax.axis_index("c")
    s = lax.axis_index("s")
    tid = c * NS + s

    zero16 = jnp.zeros((16,), jnp.float32)

    def zfill(r, _):
        for j in range(DIM // 16):
            rows_v[r, pl.ds(j * 16, 16)] = zero16
        return 0

    lax.fori_loop(0, CH, zfill, 0)
    for b in range(RPT // CH):
        pltpu.sync_copy(rows_v, acc_sh.at[pl.ds(s * RPT + b * CH, CH)])
    plsc.subcore_barrier()

    def chunk(i, _):
        ci = tid * CPT + i
        pltpu.sync_copy(row2d_hbm.at[ci], ridx_v)
        pltpu.sync_copy(col2d_hbm.at[ci], cidx_v)
        pltpu.async_copy(g_hbm.at[ridx_v], rows_v, sem).wait()
        pltpu.sync_copy(rows_v, acc_sh.at[cidx_v], add=True)
        return 0

    lax.fori_loop(0, CPT, chunk, 0)
    plsc.subcore_barrier()
    pltpu.sync_copy(acc_sh.at[pl.ds(s * RPT, RPT)],
                    out_hbm.at[c, pl.ds(s * RPT, RPT)])


def _make_sc_kernels(interpret=False):
    deg = pl.kernel(
        _sc_degree_body,
        out_type=jax.ShapeDtypeStruct((NC, NP, HL), jnp.float32),
        mesh=_SC_MESH,
        scratch_types=[
            pltpu.VMEM((CH, HL), jnp.float32),
            pltpu.VMEM((CH,), jnp.int32),
            pltpu.VMEM_SHARED((NP, HL), jnp.float32),
        ],
        interpret=interpret,
    )
    prop = pl.kernel(
        _sc_propagate_body,
        out_type=jax.ShapeDtypeStruct((NC, NP, DIM), jnp.float32),
        mesh=_SC_MESH,
        scratch_types=[
            pltpu.VMEM((CH, DIM), jnp.float32),
            pltpu.VMEM((CH,), jnp.int32),
            pltpu.VMEM((CH,), jnp.int32),
            pltpu.VMEM_SHARED((NP, DIM), jnp.float32),
            pltpu.SemaphoreType.DMA,
        ],
        interpret=interpret,
    )
    return deg, prop


_sc_degree, _sc_propagate = _make_sc_kernels()


RB = 2000


def _deg_dis(hist):
    deg = hist[0] + hist[1] + 1.0
    return lax.rsqrt(deg)


def _tc_init_body(hist_ref, x_ref, g_ref):
    dis = _deg_dis(hist_ref[...])
    g_ref[...] = dis * x_ref[...]


def _tc_update_body(hist_ref, s_ref, g_ref, x_ref, o_ref):
    dis = _deg_dis(hist_ref[...])
    su = s_ref[0] + s_ref[1] + g_ref[...]
    o_ref[...] = ((1.0 - TELEPORT) * dis * dis) * su + (TELEPORT * dis) * x_ref[...]


def _tc_final_body(hist_ref, s_ref, g_ref, x_ref, o_ref):
    dis = _deg_dis(hist_ref[...])
    su = s_ref[0] + s_ref[1] + g_ref[...]
    o_ref[...] = ((1.0 - TELEPORT) * dis) * su + TELEPORT * x_ref[...]


_HIST_SPEC = pl.BlockSpec((NC, RB, HL), lambda i: (0, i, 0))
_S_SPEC = pl.BlockSpec((NC, RB, DIM), lambda i: (0, i, 0))
_ROW_SPEC = pl.BlockSpec((RB, DIM), lambda i: (i, 0))
_OUT_SDS = jax.ShapeDtypeStruct((N_NODES, DIM), jnp.float32)
_GRID = (N_NODES // RB,)

_tc_init = pl.pallas_call(
    _tc_init_body,
    grid=_GRID,
    in_specs=[_HIST_SPEC, _ROW_SPEC],
    out_specs=_ROW_SPEC,
    out_shape=_OUT_SDS,
)

_tc_update = pl.pallas_call(
    _tc_update_body,
    grid=_GRID,
    in_specs=[_HIST_SPEC, _S_SPEC, _ROW_SPEC, _ROW_SPEC],
    out_specs=_ROW_SPEC,
    out_shape=_OUT_SDS,
)

_tc_final = pl.pallas_call(
    _tc_final_body,
    grid=_GRID,
    in_specs=[_HIST_SPEC, _S_SPEC, _ROW_SPEC, _ROW_SPEC],
    out_specs=_ROW_SPEC,
    out_shape=_OUT_SDS,
)


def kernel(x, edge_index):
    row = edge_index[0]
    col = edge_index[1]
    pad = EPAD - NUM_EDGES
    rowp = jnp.concatenate([row, jnp.zeros((pad,), row.dtype)]).reshape(NCHUNKS, CH)
    colp = jnp.concatenate([col, jnp.full((pad,), N_NODES, col.dtype)]).reshape(NCHUNKS, CH)

    hist = _sc_degree(colp)
    g = _tc_init(hist, x)
    for it in range(NUM_ITERS):
        s_parts = _sc_propagate(rowp, colp, g)
        if it < NUM_ITERS - 1:
            g = _tc_update(hist, s_parts, g, x)
        else:
            g = _tc_final(hist, s_parts, g, x)
    return g


# speedup vs baseline: 12.8591x; 1.6230x over previous
import functools

import jax
import jax.numpy as jnp
from jax import lax
from jax.experimental import pallas as pl
from jax.experimental.pallas import tpu as pltpu
from jax.experimental.pallas import tpu_sc as plsc

N_NODES = 10000
DIM = 128
NUM_EDGES = 320000
NUM_ITERS = 5
TELEPORT = 0.8

NC = 2
NS = 16
NW = NC * NS
CH = 96
NCHUNKS = 3360
EPAD = NCHUNKS * CH
CPT = NCHUNKS // NW
NBUF = 2
NP = 10240
RPT = NP // NS
HL = 128

_SC_MESH = plsc.VectorSubcoreMesh(
    core_axis_name="c", subcore_axis_name="s", num_cores=NC, num_subcores=NS)


def _sc_degree_body(col2d_hbm, hist_hbm, src_v, cidx_all, hist_sh, dsem):
    c = lax.axis_index("c")
    s = lax.axis_index("s")
    tid = c * NS + s

    zero16 = jnp.zeros((16,), jnp.float32)

    def zfill(r, _):
        for j in range(HL // 16):
            src_v[r, pl.ds(j * 16, 16)] = zero16
        return 0

    lax.fori_loop(0, CH, zfill, 0)
    nfull = RPT // CH
    for b in range(nfull):
        pltpu.sync_copy(src_v, hist_sh.at[pl.ds(s * RPT + b * CH, CH)])
    rem = RPT - nfull * CH
    if rem:
        pltpu.sync_copy(src_v.at[pl.ds(0, rem)],
                        hist_sh.at[pl.ds(s * RPT + nfull * CH, rem)])
    plsc.subcore_barrier()

    ones16 = jnp.ones((16,), jnp.float32)

    def ofill(r, _):
        for j in range(HL // 16):
            src_v[r, pl.ds(j * 16, 16)] = ones16
        return 0

    lax.fori_loop(0, CH, ofill, 0)
    pltpu.sync_copy(col2d_hbm.at[tid], cidx_all)

    def fire(i, _):
        pltpu.async_copy(src_v, hist_sh.at[cidx_all.at[i]], dsem, add=True)
        return 0

    lax.fori_loop(0, CPT, fire, 0)

    def drain(i, _):
        pltpu.make_async_copy(src_v, hist_sh.at[cidx_all.at[0]], dsem).wait()
        return 0

    lax.fori_loop(0, CPT, drain, 0)
    plsc.subcore_barrier()
    pltpu.sync_copy(hist_sh.at[pl.ds(s * RPT, RPT)],
                    hist_hbm.at[c, pl.ds(s * RPT, RPT)])


def _sc_propagate_body(row2d_hbm, col2d_hbm, g_hbm, out_hbm,
                       ridx_all, cidx_all, rb0, rb1, acc_sh, gs0, gs1):
    c = lax.axis_index("c")
    s = lax.axis_index("s")
    tid = c * NS + s
    bufs = [rb0, rb1]
    gsems = [gs0, gs1]

    zero16 = jnp.zeros((16,), jnp.float32)

    def zfill(r, _):
        for j in range(DIM // 16):
            rb0[r, pl.ds(j * 16, 16)] = zero16
        return 0

    lax.fori_loop(0, CH, zfill, 0)
    nfull = RPT // CH
    for b in range(nfull):
        pltpu.sync_copy(rb0, acc_sh.at[pl.ds(s * RPT + b * CH, CH)])
    rem = RPT - nfull * CH
    if rem:
        pltpu.sync_copy(rb0.at[pl.ds(0, rem)],
                        acc_sh.at[pl.ds(s * RPT + nfull * CH, rem)])
    plsc.subcore_barrier()

    pltpu.sync_copy(row2d_hbm.at[tid], ridx_all)
    pltpu.sync_copy(col2d_hbm.at[tid], cidx_all)

    def fire_gather(i, u):
        pltpu.async_copy(g_hbm.at[ridx_all.at[pl.ds(i * CH, CH)]],
                         bufs[u], gsems[u])

    def wait_gather(u):
        pltpu.make_async_copy(g_hbm.at[ridx_all.at[pl.ds(0, CH)]], bufs[u],
                              gsems[u]).wait()

    def scatter(i, u):
        pltpu.sync_copy(bufs[u], acc_sh.at[cidx_all.at[i]], add=True)

    fire_gather(0, 0)
    fire_gather(1, 1)

    def body(k, _):
        for u in range(2):
            i = 2 * k + u
            wait_gather(u)
            scatter(i, u)
            fire_gather(i + 2, u)
        return 0

    lax.fori_loop(0, (CPT - 3) // 2, body, 0)
    for i in range(CPT - 3, CPT):
        u = i % 2
        wait_gather(u)
        scatter(i, u)
        if i + 2 < CPT:
            fire_gather(i + 2, u)
    plsc.subcore_barrier()
    pltpu.sync_copy(acc_sh.at[pl.ds(s * RPT, RPT)],
                    out_hbm.at[c, pl.ds(s * RPT, RPT)])


def _make_sc_kernels(interpret=False):
    deg = pl.kernel(
        _sc_degree_body,
        out_type=jax.ShapeDtypeStruct((NC, NP, HL), jnp.float32),
        mesh=_SC_MESH,
        scratch_types=[
            pltpu.VMEM((CH, HL), jnp.float32),
            pltpu.VMEM((CPT, CH), jnp.int32),
            pltpu.VMEM_SHARED((NP, HL), jnp.float32),
            pltpu.SemaphoreType.DMA,
        ],
        interpret=interpret,
    )
    prop = pl.kernel(
        _sc_propagate_body,
        out_type=jax.ShapeDtypeStruct((NC, NP, DIM), jnp.float32),
        mesh=_SC_MESH,
        scratch_types=[
            pltpu.VMEM((CPT * CH,), jnp.int32),
            pltpu.VMEM((CPT, CH), jnp.int32),
        ] + [pltpu.VMEM((CH, DIM), jnp.float32)] * NBUF + [
            pltpu.VMEM_SHARED((NP, DIM), jnp.float32),
        ] + [pltpu.SemaphoreType.DMA] * NBUF,
        interpret=interpret,
    )
    return deg, prop


_sc_degree, _sc_propagate = _make_sc_kernels()


RB = 2000


def _deg_dis(hist):
    deg = hist[0] + hist[1] + 1.0
    return lax.rsqrt(deg)


def _tc_init_body(hist_ref, x_ref, g_ref):
    dis = _deg_dis(hist_ref[...])
    g_ref[...] = dis * x_ref[...]


def _tc_update_body(hist_ref, s_ref, g_ref, x_ref, o_ref):
    dis = _deg_dis(hist_ref[...])
    su = s_ref[0] + s_ref[1] + g_ref[...]
    o_ref[...] = ((1.0 - TELEPORT) * dis * dis) * su + (TELEPORT * dis) * x_ref[...]


def _tc_final_body(hist_ref, s_ref, g_ref, x_ref, o_ref):
    dis = _deg_dis(hist_ref[...])
    su = s_ref[0] + s_ref[1] + g_ref[...]
    o_ref[...] = ((1.0 - TELEPORT) * dis) * su + TELEPORT * x_ref[...]


_HIST_SPEC = pl.BlockSpec((NC, RB, HL), lambda i: (0, i, 0))
_S_SPEC = pl.BlockSpec((NC, RB, DIM), lambda i: (0, i, 0))
_ROW_SPEC = pl.BlockSpec((RB, DIM), lambda i: (i, 0))
_OUT_SDS = jax.ShapeDtypeStruct((N_NODES, DIM), jnp.float32)
_GRID = (N_NODES // RB,)

_tc_init = pl.pallas_call(
    _tc_init_body,
    grid=_GRID,
    in_specs=[_HIST_SPEC, _ROW_SPEC],
    out_specs=_ROW_SPEC,
    out_shape=_OUT_SDS,
)

_tc_update = pl.pallas_call(
    _tc_update_body,
    grid=_GRID,
    in_specs=[_HIST_SPEC, _S_SPEC, _ROW_SPEC, _ROW_SPEC],
    out_specs=_ROW_SPEC,
    out_shape=_OUT_SDS,
)

_tc_final = pl.pallas_call(
    _tc_final_body,
    grid=_GRID,
    in_specs=[_HIST_SPEC, _S_SPEC, _ROW_SPEC, _ROW_SPEC],
    out_specs=_ROW_SPEC,
    out_shape=_OUT_SDS,
)


def kernel(x, edge_index):
    row = edge_index[0]
    col = edge_index[1]
    pad = EPAD - NUM_EDGES
    rowp = jnp.concatenate([row, jnp.zeros((pad,), row.dtype)]).reshape(NW, CPT * CH)
    colp = jnp.concatenate([col, jnp.full((pad,), N_NODES, col.dtype)]).reshape(NW, CPT, CH)

    hist = _sc_degree(colp)
    g = _tc_init(hist, x)
    for it in range(NUM_ITERS):
        s_parts = _sc_propagate(rowp, colp, g)
        if it < NUM_ITERS - 1:
            g = _tc_update(hist, s_parts, g, x)
        else:
            g = _tc_final(hist, s_parts, g, x)
    return g


# speedup vs baseline: 13.6716x; 1.0632x over previous
import functools

import jax
import jax.numpy as jnp
from jax import lax
from jax.experimental import pallas as pl
from jax.experimental.pallas import tpu as pltpu
from jax.experimental.pallas import tpu_sc as plsc

N_NODES = 10000
DIM = 128
NUM_EDGES = 320000
NUM_ITERS = 5
TELEPORT = 0.8

NC = 2
NS = 16
NW = NC * NS
CH = 96
NCHUNKS = 3360
EPAD = NCHUNKS * CH
CPT = NCHUNKS // NW
NBUF = 3
NP = 10240
RPT = NP // NS
HL = 128

_SC_MESH = plsc.VectorSubcoreMesh(
    core_axis_name="c", subcore_axis_name="s", num_cores=NC, num_subcores=NS)


def _sc_degree_body(col2d_hbm, hist_hbm, src_v, cidx_all, hist_sh, dsem):
    c = lax.axis_index("c")
    s = lax.axis_index("s")
    tid = c * NS + s

    zero16 = jnp.zeros((16,), jnp.float32)

    def zfill(r, _):
        for j in range(HL // 16):
            src_v[r, pl.ds(j * 16, 16)] = zero16
        return 0

    lax.fori_loop(0, CH, zfill, 0)
    nfull = RPT // CH
    for b in range(nfull):
        pltpu.sync_copy(src_v, hist_sh.at[pl.ds(s * RPT + b * CH, CH)])
    rem = RPT - nfull * CH
    if rem:
        pltpu.sync_copy(src_v.at[pl.ds(0, rem)],
                        hist_sh.at[pl.ds(s * RPT + nfull * CH, rem)])
    plsc.subcore_barrier()

    ones16 = jnp.ones((16,), jnp.float32)

    def ofill(r, _):
        for j in range(HL // 16):
            src_v[r, pl.ds(j * 16, 16)] = ones16
        return 0

    lax.fori_loop(0, CH, ofill, 0)
    pltpu.sync_copy(col2d_hbm.at[tid], cidx_all)

    def fire(i, _):
        pltpu.async_copy(src_v, hist_sh.at[cidx_all.at[i]], dsem, add=True)
        return 0

    lax.fori_loop(0, CPT, fire, 0)

    def drain(i, _):
        pltpu.make_async_copy(src_v, hist_sh.at[cidx_all.at[0]], dsem).wait()
        return 0

    lax.fori_loop(0, CPT, drain, 0)
    plsc.subcore_barrier()
    pltpu.sync_copy(hist_sh.at[pl.ds(s * RPT, RPT)],
                    hist_hbm.at[c, pl.ds(s * RPT, RPT)])


def _sc_propagate_body(row4d_hbm, col4d_hbm, g_hbm, out_hbm,
                       xb0, xb1, xb2, cb0, cb1, cb2, rb0, rb1, rb2, acc_sh,
                       gs0, gs1, gs2, ss0, ss1, ss2, cs0, cs1, cs2,
                       xs0, xs1, xs2):
    c = lax.axis_index("c")
    s = lax.axis_index("s")
    tid = c * NS + s
    xbufs = [xb0, xb1, xb2]
    cbufs = [cb0, cb1, cb2]
    bufs = [rb0, rb1, rb2]
    gsems = [gs0, gs1, gs2]
    ssems = [ss0, ss1, ss2]
    csems = [cs0, cs1, cs2]
    xsems = [xs0, xs1, xs2]

    zero16 = jnp.zeros((16,), jnp.float32)

    def zfill(r, _):
        for j in range(DIM // 16):
            rb0[r, pl.ds(j * 16, 16)] = zero16
        return 0

    lax.fori_loop(0, CH, zfill, 0)
    nfull = RPT // CH
    for b in range(nfull):
        pltpu.sync_copy(rb0, acc_sh.at[pl.ds(s * RPT + b * CH, CH)])
    rem = RPT - nfull * CH
    if rem:
        pltpu.sync_copy(rb0.at[pl.ds(0, rem)],
                        acc_sh.at[pl.ds(s * RPT + nfull * CH, rem)])
    plsc.subcore_barrier()

    def fire_x(i, u):
        pltpu.async_copy(row4d_hbm.at[tid, i], xbufs[u], xsems[u])

    def wait_x(u):
        pltpu.make_async_copy(row4d_hbm.at[tid, 0], xbufs[u], xsems[u]).wait()

    def fire_c(i, u):
        pltpu.async_copy(col4d_hbm.at[tid, i], cbufs[u], csems[u])

    def wait_c(u):
        pltpu.make_async_copy(col4d_hbm.at[tid, 0], cbufs[u], csems[u]).wait()

    def fire_g(u):
        pltpu.async_copy(g_hbm.at[xbufs[u].at[0]], bufs[u], gsems[u])

    def wait_g(u):
        pltpu.make_async_copy(g_hbm.at[xbufs[u].at[0]], bufs[u],
                              gsems[u]).wait()

    def fire_s(u):
        pltpu.async_copy(bufs[u], acc_sh.at[cbufs[u].at[0]], ssems[u],
                         add=True)

    def wait_s(u):
        pltpu.make_async_copy(bufs[u], acc_sh.at[cbufs[u].at[0]],
                              ssems[u]).wait()

    def step(i, d, first, fire3, fire2):
        u = d
        v = (d + 2) % 3
        wait_g(u)
        wait_c(u)
        fire_s(u)
        if fire3:
            fire_x(i + 3, u)
        if fire2:
            if not first:
                wait_s(v)
            wait_x(v)
            fire_g(v)
            fire_c(i + 2, v)

    fire_c(0, 0)
    fire_c(1, 1)
    fire_x(0, 0)
    fire_x(1, 1)
    fire_x(2, 2)
    wait_x(0)
    fire_g(0)
    wait_x(1)
    fire_g(1)

    step(0, 0, True, True, True)
    step(1, 1, False, True, True)
    step(2, 2, False, True, True)

    def body(k, _):
        base = 3 * k
        step(base + 0, 0, False, True, True)
        step(base + 1, 1, False, True, True)
        step(base + 2, 2, False, True, True)
        return 0

    lax.fori_loop(1, CPT // 3 - 1, body, 0)

    base = CPT - 3
    step(base + 0, 0, False, False, True)
    step(base + 1, 1, False, False, False)
    step(base + 2, 2, False, False, False)
    for u in range(3):
        wait_s(u)
    plsc.subcore_barrier()
    pltpu.sync_copy(acc_sh.at[pl.ds(s * RPT, RPT)],
                    out_hbm.at[c, pl.ds(s * RPT, RPT)])


def _make_sc_kernels(interpret=False):
    deg = pl.kernel(
        _sc_degree_body,
        out_type=jax.ShapeDtypeStruct((NC, NP, HL), jnp.float32),
        mesh=_SC_MESH,
        scratch_types=[
            pltpu.VMEM((CH, HL), jnp.float32),
            pltpu.VMEM((CPT, CH), jnp.int32),
            pltpu.VMEM_SHARED((NP, HL), jnp.float32),
            pltpu.SemaphoreType.DMA,
        ],
        interpret=interpret,
    )
    prop = pl.kernel(
        _sc_propagate_body,
        out_type=jax.ShapeDtypeStruct((NC, NP, DIM), jnp.float32),
        mesh=_SC_MESH,
        scratch_types=[
        ] + [pltpu.VMEM((1, CH), jnp.int32)] * 6 + [
            pltpu.VMEM((CH, DIM), jnp.float32)] * 3 + [
            pltpu.VMEM_SHARED((NP, DIM), jnp.float32),
        ] + [pltpu.SemaphoreType.DMA] * 12,
        interpret=interpret,
    )
    return deg, prop


_sc_degree, _sc_propagate = _make_sc_kernels()


RB = 2000


def _deg_dis(hist):
    deg = hist[0] + hist[1] + 1.0
    return lax.rsqrt(deg)


def _tc_init_body(hist_ref, x_ref, g_ref):
    dis = _deg_dis(hist_ref[...])
    g_ref[...] = dis * x_ref[...]


def _tc_update_body(hist_ref, s_ref, g_ref, x_ref, o_ref):
    dis = _deg_dis(hist_ref[...])
    su = s_ref[0] + s_ref[1] + g_ref[...]
    o_ref[...] = ((1.0 - TELEPORT) * dis * dis) * su + (TELEPORT * dis) * x_ref[...]


def _tc_final_body(hist_ref, s_ref, g_ref, x_ref, o_ref):
    dis = _deg_dis(hist_ref[...])
    su = s_ref[0] + s_ref[1] + g_ref[...]
    o_ref[...] = ((1.0 - TELEPORT) * dis) * su + TELEPORT * x_ref[...]


_HIST_SPEC = pl.BlockSpec((NC, RB, HL), lambda i: (0, i, 0))
_S_SPEC = pl.BlockSpec((NC, RB, DIM), lambda i: (0, i, 0))
_ROW_SPEC = pl.BlockSpec((RB, DIM), lambda i: (i, 0))
_OUT_SDS = jax.ShapeDtypeStruct((N_NODES, DIM), jnp.float32)
_GRID = (N_NODES // RB,)

_tc_init = pl.pallas_call(
    _tc_init_body,
    grid=_GRID,
    in_specs=[_HIST_SPEC, _ROW_SPEC],
    out_specs=_ROW_SPEC,
    out_shape=_OUT_SDS,
)

_tc_update = pl.pallas_call(
    _tc_update_body,
    grid=_GRID,
    in_specs=[_HIST_SPEC, _S_SPEC, _ROW_SPEC, _ROW_SPEC],
    out_specs=_ROW_SPEC,
    out_shape=_OUT_SDS,
)

_tc_final = pl.pallas_call(
    _tc_final_body,
    grid=_GRID,
    in_specs=[_HIST_SPEC, _S_SPEC, _ROW_SPEC, _ROW_SPEC],
    out_specs=_ROW_SPEC,
    out_shape=_OUT_SDS,
)


def kernel(x, edge_index):
    row = edge_index[0]
    col = edge_index[1]
    pad = EPAD - NUM_EDGES
    rowp = jnp.concatenate([row, jnp.zeros((pad,), row.dtype)]).reshape(NW, CPT, 1, CH)
    colp = jnp.concatenate([col, jnp.full((pad,), N_NODES, col.dtype)])
    colp2 = colp.reshape(NW, CPT, CH)
    colp4 = colp.reshape(NW, CPT, 1, CH)

    hist = _sc_degree(colp2)
    g = _tc_init(hist, x)
    for it in range(NUM_ITERS):
        s_parts = _sc_propagate(rowp, colp4, g)
        if it < NUM_ITERS - 1:
            g = _tc_update(hist, s_parts, g, x)
        else:
            g = _tc_final(hist, s_parts, g, x)
    return g
